# Initial kernel scaffold; baseline (speedup 1.0000x reference)
import jax
import jax.numpy as jnp
from jax.experimental import pallas as pl


def kernel(users, items, emb_user, emb_item, edge_rows, edge_cols, edge_vals):
    raise NotImplementedError("write your pallas kernel here")


# speedup vs baseline: 5.9048x; 5.9048x over previous
import functools

import jax
import jax.numpy as jnp
from jax import lax
from jax.experimental import pallas as pl
from jax.experimental.pallas import tpu as pltpu
from jax.experimental.pallas import tpu_sc as plsc

N_U = 30000
N_NODES = 50000
D = 64
DH = 32
E_TOT = 800000
BATCH = 4096

K = 640
G = K // 128
NCH = E_TOT // K
JMAX = (NCH + 15) // 16
C = 128
NFULL = N_NODES // C
TAIL = N_NODES - NFULL * C
BPT = BATCH // 32

_MESH = plsc.VectorSubcoreMesh(
    core_axis_name="c", subcore_axis_name="s", num_cores=2, num_subcores=16
)
_PARAMS = pltpu.CompilerParams(use_tc_tiling_on_sc=False,
                               needs_layout_passes=False)


def _zero_rows(buf, nrows, ngroups):
    z = jnp.zeros((16,), jnp.float32)

    def body(i, carry):
        for g in range(ngroups):
            buf[i, pl.ds(g * 16, 16)] = z
        return carry

    lax.fori_loop(0, nrows, body, 0)


@functools.partial(
    pl.kernel,
    out_type=(
        jax.ShapeDtypeStruct((N_NODES, DH), jnp.float32),
        jax.ShapeDtypeStruct((N_NODES, DH), jnp.float32),
    ),
    mesh=_MESH,
    scratch_types=[
        pltpu.VMEM_SHARED((N_NODES, DH), jnp.float32),
        pltpu.VMEM((K,), jnp.int32),
        pltpu.VMEM((G, 128), jnp.int32),
        pltpu.VMEM((K,), jnp.int32),
        pltpu.VMEM((K,), jnp.float32),
        pltpu.VMEM((K, DH), jnp.float32),
        pltpu.VMEM((C, DH), jnp.float32),
        pltpu.SemaphoreType.DMA,
    ],
    compiler_params=_PARAMS,
)
def _layer(ein_a, ein_b, rows, cols, vals, eout_a, eout_b, acc, rows_vl,
           rows_v, cols_v, vals_v, gbuf, dbuf, sem):
    c = lax.axis_index("c")
    s = lax.axis_index("s")

    _zero_rows(dbuf, C, DH // 16)

    def zbody(j, carry):
        t = s + 16 * j

        @pl.when(t < NFULL)
        def _():
            pltpu.sync_copy(dbuf, acc.at[pl.ds(t * C, C)])

        return carry

    lax.fori_loop(0, (NFULL + 16) // 16, zbody, 0)

    @pl.when(s == 0)
    def _():
        pltpu.sync_copy(dbuf.at[pl.ds(0, TAIL)],
                        acc.at[pl.ds(NFULL * C, TAIL)])

    plsc.subcore_barrier()

    def edge_pass(ein):
        def chunk(j, carry):
            t = s + 16 * j

            @pl.when(t < NCH)
            def _():
                base = t * K
                pltpu.sync_copy(rows.at[pl.ds(base, K)], rows_vl)
                pltpu.sync_copy(cols.at[pl.ds(base, K)], cols_v)
                pltpu.sync_copy(vals.at[pl.ds(base, K)], vals_v)

                def fix(i, carry2):
                    for g in range(G):
                        rows_v[g, pl.ds(i * 16, 16)] = (
                            rows_vl[pl.ds(g * 128 + i * 16, 16)]
                        )
                    return carry2

                lax.fori_loop(0, 8, fix, 0)

                for g in range(G):
                    pltpu.async_copy(ein.at[cols_v.at[pl.ds(g * 128, 128)]],
                                     gbuf.at[pl.ds(g * 128, 128)], sem).wait()

                def mul(i, carry2):
                    vv = vals_v[pl.ds(i * 16, 16)]
                    for u in range(16):
                        ee = i * 16 + u
                        bv = jnp.broadcast_to(vv[u], (16,))
                        for g in range(DH // 16):
                            gbuf[ee, pl.ds(g * 16, 16)] = (
                                gbuf[ee, pl.ds(g * 16, 16)] * bv
                            )
                    return carry2

                lax.fori_loop(0, K // 16, mul, 0)

                for g in range(G):
                    pltpu.sync_copy(gbuf.at[pl.ds(g * 128, 128)],
                                    acc.at[rows_v.at[g]], add=True)

            return carry

        lax.fori_loop(0, JMAX, chunk, 0)

    @pl.when(c == 0)
    def _():
        edge_pass(ein_a)

    @pl.when(c == 1)
    def _():
        edge_pass(ein_b)

    plsc.subcore_barrier()

    def drain_to(eout):
        def drain(j, carry):
            t = s + 16 * j

            @pl.when(t < NFULL)
            def _():
                pltpu.sync_copy(acc.at[pl.ds(t * C, C)], dbuf)
                pltpu.sync_copy(dbuf, eout.at[pl.ds(t * C, C)])

            return carry

        lax.fori_loop(0, (NFULL + 16) // 16, drain, 0)

        @pl.when(s == 1)
        def _():
            pltpu.sync_copy(acc.at[pl.ds(NFULL * C, TAIL)],
                            dbuf.at[pl.ds(0, TAIL)])
            pltpu.sync_copy(dbuf.at[pl.ds(0, TAIL)],
                            eout.at[pl.ds(NFULL * C, TAIL)])

    @pl.when(c == 0)
    def _():
        drain_to(eout_a)

    @pl.when(c == 1)
    def _():
        drain_to(eout_b)


@functools.partial(
    pl.kernel,
    out_type=jax.ShapeDtypeStruct((BATCH,), jnp.float32),
    mesh=_MESH,
    scratch_types=[
        pltpu.VMEM((BPT,), jnp.int32),
        pltpu.VMEM((BPT,), jnp.int32),
        pltpu.VMEM((BPT, DH), jnp.float32),
        pltpu.VMEM((BPT, DH), jnp.float32),
        pltpu.VMEM((BPT, DH), jnp.float32),
        pltpu.VMEM((BPT, DH), jnp.float32),
        pltpu.VMEM((BPT, DH), jnp.float32),
        pltpu.VMEM((BPT,), jnp.float32),
        pltpu.SemaphoreType.DMA,
    ],
    compiler_params=_PARAMS,
)
def _final(users, items, e0a, e0b, e1a, e1b, e2a, e2b, e3a, e3b, gamma,
           uidx, iidx, uacc_a, uacc_b, iacc_a, iacc_b, gb, gout, sem):
    c = lax.axis_index("c")
    s = lax.axis_index("s")
    w = s * 2 + c
    bbase = w * BPT
    pltpu.sync_copy(users.at[pl.ds(bbase, BPT)], uidx)
    pltpu.sync_copy(items.at[pl.ds(bbase, BPT)], iidx)
    off = jnp.broadcast_to(N_U, (16,)).astype(jnp.int32)

    def fix(i, carry):
        iidx[pl.ds(i * 16, 16)] = iidx[pl.ds(i * 16, 16)] + off
        return carry

    lax.fori_loop(0, BPT // 16, fix, 0)

    pltpu.async_copy(e0a.at[uidx], uacc_a, sem).wait()
    pltpu.async_copy(e0b.at[uidx], uacc_b, sem).wait()
    pltpu.async_copy(e0a.at[iidx], iacc_a, sem).wait()
    pltpu.async_copy(e0b.at[iidx], iacc_b, sem).wait()

    def accumulate(ek, dst, idx):
        pltpu.async_copy(ek.at[idx], gb, sem).wait()

        def add(i, carry):
            for g in range(DH // 16):
                dst[i, pl.ds(g * 16, 16)] = (
                    dst[i, pl.ds(g * 16, 16)] + gb[i, pl.ds(g * 16, 16)]
                )
            return carry

        lax.fori_loop(0, BPT, add, 0)

    for eka, ekb in ((e1a, e1b), (e2a, e2b), (e3a, e3b)):
        accumulate(eka, uacc_a, uidx)
        accumulate(ekb, uacc_b, uidx)
        accumulate(eka, iacc_a, iidx)
        accumulate(ekb, iacc_b, iidx)

    lanes = lax.iota(jnp.int32, 16)

    def dot16(i, carry):
        res = jnp.zeros((16,), jnp.float32)
        for u in range(16):
            e = i * 16 + u
            p = uacc_a[e, pl.ds(0, 16)] * iacc_a[e, pl.ds(0, 16)]
            p = p + uacc_a[e, pl.ds(16, 16)] * iacc_a[e, pl.ds(16, 16)]
            p = p + uacc_b[e, pl.ds(0, 16)] * iacc_b[e, pl.ds(0, 16)]
            p = p + uacc_b[e, pl.ds(16, 16)] * iacc_b[e, pl.ds(16, 16)]
            t = jnp.sum(p) * 0.0625
            res = jnp.where(lanes == u, jnp.broadcast_to(t, (16,)), res)
        gout[pl.ds(i * 16, 16)] = res
        return carry

    lax.fori_loop(0, BPT // 16, dot16, 0)
    pltpu.sync_copy(gout, gamma.at[pl.ds(bbase, BPT)])


def kernel(users, items, emb_user, emb_item, edge_rows, edge_cols, edge_vals):
    e0 = jnp.concatenate([emb_user, emb_item], axis=0)
    e0a = e0[:, :DH]
    e0b = e0[:, DH:]
    e1a, e1b = _layer(e0a, e0b, edge_rows, edge_cols, edge_vals)
    e2a, e2b = _layer(e1a, e1b, edge_rows, edge_cols, edge_vals)
    e3a, e3b = _layer(e2a, e2b, edge_rows, edge_cols, edge_vals)
    return _final(users, items, e0a, e0b, e1a, e1b, e2a, e2b, e3a, e3b)
